# baseline (speedup 1.0000x reference)
import functools

import jax
import jax.numpy as jnp
from jax import lax
from jax.experimental import pallas as pl
from jax.experimental.pallas import tpu as pltpu
from jax.experimental.pallas import tpu_sc as plsc

_N_ROWS = 4096
_N_COLS = 8192
_LANES = 16
_NW = 32
_ROWS_PER_W = _N_ROWS // _NW
_CHUNK_ROWS = 64
_SUBGROUPS = _CHUNK_ROWS // _LANES
_CHUNK_COLS = 512
_N_STAGES = _N_COLS // _CHUNK_COLS
_N_CHUNKS = _ROWS_PER_W // _CHUNK_ROWS


def _sc_body(x_hbm, out_hbm, buf):
    core = lax.axis_index("c")
    sub = lax.axis_index("s")
    wid = sub * 2 + core
    row_base = wid * _ROWS_PER_W

    row_iota = lax.iota(jnp.int32, _LANES)
    zeros = jnp.zeros((_LANES,), jnp.float32)
    group_rows = [row_iota + g * _LANES for g in range(_SUBGROUPS)]

    for chunk in range(_N_CHUNKS):
        r0 = row_base + chunk * _CHUNK_ROWS
        accs = (zeros,) * _SUBGROUPS
        for stage in range(_N_STAGES):
            c0 = stage * _CHUNK_COLS
            pltpu.sync_copy(
                x_hbm.at[pl.ds(r0, _CHUNK_ROWS), pl.ds(c0, _CHUNK_COLS)], buf)

            col0 = jnp.zeros((_LANES,), jnp.int32)

            def step(j, carry):
                colv = carry[0]
                accs_in = carry[1:]
                new_accs = []
                for g in range(_SUBGROUPS):
                    v = plsc.load_gather(buf, [group_rows[g], colv])
                    a = accs_in[g] + v
                    plsc.store_scatter(buf, [group_rows[g], colv], a)
                    new_accs.append(a)
                return (colv + 1,) + tuple(new_accs)

            carry = lax.fori_loop(0, _CHUNK_COLS, step, (col0,) + accs)
            accs = carry[1:]

            pltpu.sync_copy(
                buf, out_hbm.at[pl.ds(r0, _CHUNK_ROWS), pl.ds(c0, _CHUNK_COLS)])


@jax.jit
def kernel(x):
    mesh = plsc.VectorSubcoreMesh(core_axis_name="c", subcore_axis_name="s")
    k = functools.partial(
        pl.kernel,
        mesh=mesh,
        out_type=jax.ShapeDtypeStruct((_N_ROWS, _N_COLS), jnp.float32),
        scratch_types=[pltpu.VMEM((_CHUNK_ROWS, _CHUNK_COLS), jnp.float32)],
        compiler_params=pltpu.CompilerParams(
            use_tc_tiling_on_sc=False, needs_layout_passes=False),
    )(_sc_body)
    return k(x)


# speedup vs baseline: 12.0521x; 12.0521x over previous
import functools

import jax
import jax.numpy as jnp
from jax.experimental import pallas as pl
from jax.experimental.pallas import tpu as pltpu

_LANES = 128


def _cumsum_body(x_ref, o_ref, carry_ref, *, groups):
    li = jax.lax.broadcasted_iota(jnp.int32, (_LANES, _LANES), 0)
    lj = jax.lax.broadcasted_iota(jnp.int32, (_LANES, _LANES), 1)
    scan_mat = (li <= lj).astype(jnp.float32)

    j = pl.program_id(1)

    @pl.when(j == 0)
    def _():
        carry_ref[...] = jnp.zeros_like(carry_ref)

    carry = carry_ref[...]
    for g in range(groups):
        xg = x_ref[:, g * _LANES:(g + 1) * _LANES]
        scan = jnp.dot(xg, scan_mat, preferred_element_type=jnp.float32)
        o_ref[:, g * _LANES:(g + 1) * _LANES] = scan + carry
        carry = carry + scan[:, _LANES - 1:_LANES]
    carry_ref[...] = carry


@jax.jit
def kernel(x):
    n_rows, n_cols = x.shape
    block_rows = 256
    block_cols = 2048
    grid = (n_rows // block_rows, n_cols // block_cols)
    return pl.pallas_call(
        functools.partial(_cumsum_body, groups=block_cols // _LANES),
        grid=grid,
        in_specs=[pl.BlockSpec((block_rows, block_cols), lambda i, j: (i, j))],
        out_specs=pl.BlockSpec((block_rows, block_cols), lambda i, j: (i, j)),
        out_shape=jax.ShapeDtypeStruct((n_rows, n_cols), x.dtype),
        scratch_shapes=[pltpu.VMEM((block_rows, 1), jnp.float32)],
        compiler_params=pltpu.CompilerParams(
            dimension_semantics=("parallel", "arbitrary"),
        ),
    )(x)


# speedup vs baseline: 17.5572x; 1.4568x over previous
import functools

import jax
import jax.numpy as jnp
from jax.experimental import pallas as pl
from jax.experimental.pallas import tpu as pltpu

_LANES = 128


def _cumsum_body(x_ref, o_ref, carry_ref, *, groups):
    li = jax.lax.broadcasted_iota(jnp.int32, (_LANES, _LANES), 0)
    lj = jax.lax.broadcasted_iota(jnp.int32, (_LANES, _LANES), 1)
    scan_mat = (li <= lj).astype(jnp.float32)

    j = pl.program_id(1)

    @pl.when(j == 0)
    def _():
        carry_ref[...] = jnp.zeros_like(carry_ref)

    carry = carry_ref[...]
    for g in range(groups):
        xg = x_ref[:, g * _LANES:(g + 1) * _LANES]
        scan = jnp.dot(xg, scan_mat, preferred_element_type=jnp.float32)
        o_ref[:, g * _LANES:(g + 1) * _LANES] = scan + carry
        carry = carry + scan[:, _LANES - 1:_LANES]
    carry_ref[...] = carry


@jax.jit
def kernel(x):
    n_rows, n_cols = x.shape
    block_rows = 1024
    block_cols = 2048
    grid = (n_rows // block_rows, n_cols // block_cols)
    return pl.pallas_call(
        functools.partial(_cumsum_body, groups=block_cols // _LANES),
        grid=grid,
        in_specs=[pl.BlockSpec((block_rows, block_cols), lambda i, j: (i, j))],
        out_specs=pl.BlockSpec((block_rows, block_cols), lambda i, j: (i, j)),
        out_shape=jax.ShapeDtypeStruct((n_rows, n_cols), x.dtype),
        scratch_shapes=[pltpu.VMEM((block_rows, 1), jnp.float32)],
        compiler_params=pltpu.CompilerParams(
            dimension_semantics=("parallel", "arbitrary"),
        ),
    )(x)
